# baseline (speedup 1.0000x reference)
import functools

import jax
import jax.numpy as jnp
from jax import lax
from jax.experimental import pallas as pl
from jax.experimental.pallas import tpu as pltpu
from jax.experimental.pallas import tpu_sc as plsc

NUM_CORES = 2
NUM_SUBCORES = 16
LANES = 16
NW = NUM_CORES * NUM_SUBCORES
HALF = 256


def _amf_body(b_per_w, d,
              user_h, item_h, gu_h, gi_h,
              xui_h, gout_u_h, gout_i_h,
              idx_u, idx_i, a, c, at_u, at_i, xv,
              s1, s2):
  wid = lax.axis_index("s") * NUM_CORES + lax.axis_index("c")
  base = wid * b_per_w
  dsl = pl.ds(base, b_per_w)
  nvec = d // LANES
  lanes = lax.iota(jnp.int32, LANES)
  lane_masks = [lanes == l for l in range(LANES)]

  pltpu.sync_copy(user_h.at[dsl], idx_u)
  pltpu.sync_copy(item_h.at[dsl], idx_i)

  n_half = b_per_w // HALF

  for h in range(n_half):
    hsl = pl.ds(h * HALF, HALF)
    cp_gu = pltpu.make_async_copy(gu_h.at[idx_u.at[hsl]], a, s1)
    cp_gi = pltpu.make_async_copy(gi_h.at[idx_i.at[hsl]], c, s2)
    cp_gu.start()
    cp_gi.start()
    cp_gu.wait()
    cp_gi.wait()

    def group(g, carry):
      gsl = pl.ds(g * LANES, LANES)
      xacc = jnp.zeros((LANES,), jnp.float32)
      for l in range(LANES):
        r = g * LANES + l
        rvec = jnp.full((LANES,), r, jnp.int32)
        p = None
        for j in range(nvec):
          sl = pl.ds(j * LANES, LANES)
          av = a[r, sl]
          cv = c[r, sl]
          plsc.store_scatter(at_u, [lanes + j * LANES, rvec], av)
          plsc.store_scatter(at_i, [lanes + j * LANES, rvec], cv)
          p = av * cv if p is None else p + av * cv
        tot = jnp.full((LANES,), jnp.sum(p), jnp.float32)
        xacc = jnp.where(lane_masks[l], tot, xacc)
      xv[pl.ds(h * HALF + g * LANES, LANES)] = xacc
      return carry

    lax.fori_loop(0, HALF // LANES, group, 0)

    pltpu.sync_copy(at_u, gout_u_h.at[:, pl.ds(base + h * HALF, HALF)])
    pltpu.sync_copy(at_i, gout_i_h.at[:, pl.ds(base + h * HALF, HALF)])

  pltpu.sync_copy(xv, xui_h.at[dsl])


def kernel(user, item, Bi, Gu, Gi, Delta_Gu, Delta_Gi):
  batch = user.shape[0]
  d = Gu.shape[1]
  b_per_w = batch // NW
  user = user.astype(jnp.int32)
  item = item.astype(jnp.int32)

  mesh = plsc.VectorSubcoreMesh(
      core_axis_name="c", subcore_axis_name="s",
      num_cores=NUM_CORES, num_subcores=NUM_SUBCORES)

  f32 = jnp.float32
  fn = pl.kernel(
      functools.partial(_amf_body, b_per_w, d),
      out_type=(
          jax.ShapeDtypeStruct((batch,), f32),
          jax.ShapeDtypeStruct((d, batch), f32),
          jax.ShapeDtypeStruct((d, batch), f32),
      ),
      mesh=mesh,
      compiler_params=pltpu.CompilerParams(
          needs_layout_passes=False, use_tc_tiling_on_sc=False),
      scratch_types=[
          pltpu.VMEM((b_per_w,), jnp.int32),
          pltpu.VMEM((b_per_w,), jnp.int32),
          pltpu.VMEM((HALF, d), f32),
          pltpu.VMEM((HALF, d), f32),
          pltpu.VMEM((d, HALF), f32),
          pltpu.VMEM((d, HALF), f32),
          pltpu.VMEM((b_per_w,), f32),
          pltpu.SemaphoreType.DMA,
          pltpu.SemaphoreType.DMA,
      ],
  )
  xui, gamma_u_t, gamma_i_t = fn(user, item, Gu, Gi)
  beta_i = jnp.zeros((batch,), f32)
  return (xui, beta_i, gamma_u_t.T, gamma_i_t.T)
